# baseline (speedup 1.0000x reference)
import functools

import jax
import jax.numpy as jnp
from jax import lax
from jax.experimental import pallas as pl
from jax.experimental.pallas import tpu as pltpu
from jax.experimental.pallas import tpu_sc as plsc

_BATCH = 16384
_DIM = 64
_NC = 2
_NS = 16
_NW = _NC * _NS
_BPW = _BATCH // _NW
_CHUNK = 128
_NCH = _BPW // _CHUNK
_LANES = 16
_GROUPS = _BPW // _LANES

_mesh = plsc.VectorSubcoreMesh(
    core_axis_name="c", subcore_axis_name="s", num_cores=_NC, num_subcores=_NS
)


@functools.partial(
    pl.kernel,
    out_type=jax.ShapeDtypeStruct((_BATCH,), jnp.float32),
    mesh=_mesh,
    scratch_types=[
        pltpu.VMEM((_NCH, _CHUNK), jnp.int32),
        pltpu.VMEM((_NCH, _CHUNK), jnp.int32),
        pltpu.VMEM((_BPW, _DIM), jnp.float32),
        pltpu.VMEM((_BPW, _DIM), jnp.float32),
        pltpu.VMEM((8, _LANES), jnp.float32),
        pltpu.VMEM((_BPW,), jnp.float32),
        pltpu.SemaphoreType.DMA,
    ],
    compiler_params=pltpu.CompilerParams(
        needs_layout_passes=False, use_tc_tiling_on_sc=False),
)
def _rate_similarity_sc(qidx_hbm, ridx_hbm, table_hbm, params_hbm, out_hbm,
                        qidx_v, ridx_v, qrows, rrows, params_v, out_v, sem):
    wid = lax.axis_index("s") * _NC + lax.axis_index("c")

    pltpu.sync_copy(qidx_hbm.at[pl.ds(wid * _NCH, _NCH)], qidx_v)
    pltpu.sync_copy(ridx_hbm.at[pl.ds(wid * _NCH, _NCH)], ridx_v)
    pltpu.sync_copy(params_hbm, params_v)

    copies = []
    for j in range(_NCH):
        copies.append(pltpu.async_copy(
            table_hbm.at[qidx_v.at[j]], qrows.at[pl.ds(j * _CHUNK, _CHUNK)],
            sem))
        copies.append(pltpu.async_copy(
            table_hbm.at[ridx_v.at[j]], rrows.at[pl.ds(j * _CHUNK, _CHUNK)],
            sem))
    for c in copies:
        c.wait()

    lower_v = params_v[0]
    upper_v = params_v[1]
    mid_v = params_v[2]
    rate_v = params_v[3]
    beta_v = params_v[4]

    iota = lax.iota(jnp.int32, _LANES)

    def group_body(g, carry):
        rowv = iota + g * _LANES
        acc = jnp.full((_LANES,), 1e-12, jnp.float32)
        for d in range(_DIM):
            colv = jnp.full((_LANES,), d, jnp.int32)
            qv = plsc.load_gather(qrows, [rowv, colv])
            rv = plsc.load_gather(rrows, [rowv, colv])
            df = qv - rv
            acc = acc + df * df
        seed = jnp.int32(0x5F3759DF) - (plsc.bitcast(acc, jnp.int32) >> 1)
        y = plsc.bitcast(seed, jnp.float32)
        for _ in range(3):
            y = y * (1.5 - 0.5 * acc * y * y)
        dist = acc * y
        sim = jnp.exp(-beta_v * dist)
        t = jnp.exp(-rate_v * (sim - mid_v))
        prob = lower_v + (upper_v - lower_v) / (1.0 + t)
        out_v[pl.ds(g * _LANES, _LANES)] = prob
        return carry

    lax.fori_loop(0, _GROUPS, group_body, 0)

    pltpu.sync_copy(out_v, out_hbm.at[pl.ds(wid * _BPW, _BPW)])


def kernel(stimulus_set, percept_table, lower, upper, midpoint, rate, beta):
    qidx = stimulus_set[:, 0].astype(jnp.int32).reshape(_NW * _NCH, _CHUNK)
    ridx = stimulus_set[:, 1].astype(jnp.int32).reshape(_NW * _NCH, _CHUNK)
    zero = jnp.float32(0)
    params = jnp.stack([
        jnp.float32(lower), jnp.float32(upper), jnp.float32(midpoint),
        jnp.float32(rate), jnp.float32(beta), zero, zero, zero,
    ])
    params = jnp.broadcast_to(params[:, None], (8, _LANES))
    out = _rate_similarity_sc(qidx, ridx, percept_table, params)
    return out.reshape(_BATCH, 1)
